# baseline (speedup 1.0000x reference)
import functools

import jax
import jax.numpy as jnp
from jax import lax
from jax.experimental import pallas as pl
from jax.experimental.pallas import tpu as pltpu
from jax.experimental.pallas import tpu_sc as plsc

_T = 4
_TH = _TW = 1024
_C = 4
_L = 16

_P = 2048


def _sc_sample(uv, f_mat, table):
    b, h, w, _ = uv.shape
    n = b * h * w
    info = plsc.get_sparse_core_info()
    nw = info.num_cores * info.num_subcores
    per_w = n // nw
    rows_w = per_w // w
    rpc = _P // w
    n_chunks = per_w // _P
    wsh = w.bit_length() - 1
    mesh = plsc.VectorSubcoreMesh(core_axis_name="c", subcore_axis_name="s")

    @functools.partial(
        pl.kernel,
        mesh=mesh,
        out_type=jax.ShapeDtypeStruct((b, h, w, _C), jnp.float32),
        compiler_params=pltpu.CompilerParams(
            needs_layout_passes=False, use_tc_tiling_on_sc=False),
        scratch_types=[
            pltpu.VMEM((rpc, w, 2), jnp.float32),
            pltpu.VMEM((rpc, w), jnp.int32),
            pltpu.VMEM((_P,), jnp.float32),
            pltpu.VMEM((_P,), jnp.float32),
            pltpu.VMEM((2, _P), jnp.int32),
            pltpu.VMEM((_P, 2 * _C), jnp.float32),
            pltpu.VMEM((_P, 2 * _C), jnp.float32),
            pltpu.VMEM((rpc, w, _C), jnp.float32),
            pltpu.SemaphoreType.DMA,
            pltpu.SemaphoreType.DMA,
        ],
    )
    def k(uv_hbm, fm_hbm, tab_hbm, out_hbm,
          uv_v, fm_v, fx_v, fy_v, idx_v, top_v, bot_v, o_v,
          sem_in, sem_g):
        wid = lax.axis_index("s") * info.num_cores + lax.axis_index("c")
        lanes = lax.iota(jnp.int32, _L)
        grp = lanes >> 2
        ch = lanes & 3

        def chunk_body(kc, _):
            r0 = wid * rows_w + kc * rpc
            bi = r0 // h
            hi = r0 % h
            cin = [
                pltpu.async_copy(uv_hbm.at[bi, pl.ds(hi, rpc)], uv_v, sem_in),
                pltpu.async_copy(fm_hbm.at[bi, pl.ds(hi, rpc)], fm_v, sem_in),
            ]
            for c in cin:
                c.wait()

            @plsc.parallel_loop(0, _P // _L, unroll=4)
            def _(i):
                r = i >> 5
                c0 = (i & 31) << 4
                rs = jnp.full((_L,), r, jnp.int32)
                cols = c0 + lanes
                uu = plsc.load_gather(uv_v, [rs, cols, jnp.zeros_like(lanes)])
                vv = plsc.load_gather(uv_v, [rs, cols, jnp.ones_like(lanes)])
                fm = plsc.load_gather(fm_v, [rs, cols])
                x = uu * float(_TW) - 0.5
                y = vv * float(_TH) - 0.5
                xt = x.astype(jnp.int32)
                yt = y.astype(jnp.int32)
                x0 = jnp.where(x < xt.astype(jnp.float32), xt - 1, xt)
                y0 = jnp.where(y < yt.astype(jnp.float32), yt - 1, yt)
                sl = pl.ds(i * _L, _L)
                fx_v[sl] = x - x0.astype(jnp.float32)
                fy_v[sl] = y - y0.astype(jnp.float32)
                x0w = x0 & (_TW - 1)
                base_m = (fm << 20) + x0w
                idx_v[0, sl] = base_m + ((y0 & (_TH - 1)) << 10)
                idx_v[1, sl] = base_m + (((y0 + 1) & (_TH - 1)) << 10)

            ctop = pltpu.async_copy(tab_hbm.at[idx_v.at[0]], top_v, sem_g)
            cbot = pltpu.async_copy(tab_hbm.at[idx_v.at[1]], bot_v, sem_g)
            ctop.wait()
            cbot.wait()

            @plsc.parallel_loop(0, _P // 4, unroll=4)
            def _(j):
                rows = grp + (4 * j)
                ch1 = ch + 4
                t00 = plsc.load_gather(top_v, [rows, ch])
                t01 = plsc.load_gather(top_v, [rows, ch1])
                t10 = plsc.load_gather(bot_v, [rows, ch])
                t11 = plsc.load_gather(bot_v, [rows, ch1])
                fx = plsc.load_gather(fx_v, [rows])
                fy = plsc.load_gather(fy_v, [rows])
                omx = 1.0 - fx
                top = t00 * omx + t01 * fx
                bot = t10 * omx + t11 * fx
                plsc.store_scatter(o_v, [rows >> wsh, rows & (w - 1), ch],
                                   top * (1.0 - fy) + bot * fy)

            pltpu.sync_copy(o_v, out_hbm.at[bi, pl.ds(hi, rpc)])
            return ()

        lax.fori_loop(0, n_chunks, chunk_body, ())

    return k(uv, f_mat, table)


def kernel(uv, f_mat, tex0, tex1, tex2, tex3):
    pairs = [
        jnp.concatenate([t, jnp.roll(t, -1, axis=1)], axis=-1)
        .reshape(_TH * _TW, 2 * _C)
        for t in (tex0, tex1, tex2, tex3)
    ]
    table = jnp.concatenate(pairs, axis=0)
    return _sc_sample(uv, f_mat, table)


# speedup vs baseline: 1.1011x; 1.1011x over previous
import functools

import jax
import jax.numpy as jnp
from jax import lax
from jax.experimental import pallas as pl
from jax.experimental.pallas import tpu as pltpu
from jax.experimental.pallas import tpu_sc as plsc

_T = 4
_TH = _TW = 1024
_C = 4
_L = 16

_P = 2048


def _sc_sample(uvflat, fmflat, table, b, h, w):
    n = b * h * w
    info = plsc.get_sparse_core_info()
    nw = info.num_cores * info.num_subcores
    per_w = n // nw
    rows_w = per_w // w
    rpc = _P // w
    n_chunks = per_w // _P
    wsh = w.bit_length() - 1
    mesh = plsc.VectorSubcoreMesh(core_axis_name="c", subcore_axis_name="s")

    @functools.partial(
        pl.kernel,
        mesh=mesh,
        out_type=jax.ShapeDtypeStruct((b, h, w, _C), jnp.float32),
        compiler_params=pltpu.CompilerParams(
            needs_layout_passes=False, use_tc_tiling_on_sc=False),
        scratch_types=[
            pltpu.VMEM((2 * _P,), jnp.float32),
            pltpu.VMEM((_P,), jnp.int32),
            pltpu.VMEM((_P,), jnp.float32),
            pltpu.VMEM((_P,), jnp.float32),
            pltpu.VMEM((2, _P), jnp.int32),
            pltpu.VMEM((_P, 2 * _C), jnp.float32),
            pltpu.VMEM((_P, 2 * _C), jnp.float32),
            pltpu.VMEM((rpc, w, _C), jnp.float32),
            pltpu.SemaphoreType.DMA,
            pltpu.SemaphoreType.DMA,
        ],
    )
    def k(uv_hbm, fm_hbm, tab_hbm, out_hbm,
          uv_v, fm_v, fx_v, fy_v, idx_v, top_v, bot_v, o_v,
          sem_in, sem_g):
        wid = lax.axis_index("s") * info.num_cores + lax.axis_index("c")
        lanes = lax.iota(jnp.int32, _L)
        grp = lanes >> 2
        ch = lanes & 3

        def chunk_body(kc, _):
            r0 = wid * rows_w + kc * rpc
            bi = r0 // h
            hi = r0 % h
            base = pl.multiple_of(wid * per_w + kc * _P, _P)
            cin = [
                pltpu.async_copy(uv_hbm.at[pl.ds(2 * base, 2 * _P)], uv_v,
                                 sem_in),
                pltpu.async_copy(fm_hbm.at[pl.ds(base, _P)], fm_v, sem_in),
            ]
            for c in cin:
                c.wait()

            @plsc.parallel_loop(0, _P // _L, unroll=4)
            def _(i):
                sl = pl.ds(i * _L, _L)
                pix2 = (i * (2 * _L)) + 2 * lanes
                uu = plsc.load_gather(uv_v, [pix2])
                vv = plsc.load_gather(uv_v, [pix2 + 1])
                fm = fm_v[sl]
                x = uu * float(_TW) - 0.5
                y = vv * float(_TH) - 0.5
                xt = x.astype(jnp.int32)
                yt = y.astype(jnp.int32)
                x0 = jnp.where(x < xt.astype(jnp.float32), xt - 1, xt)
                y0 = jnp.where(y < yt.astype(jnp.float32), yt - 1, yt)
                fx_v[sl] = x - x0.astype(jnp.float32)
                fy_v[sl] = y - y0.astype(jnp.float32)
                x0w = x0 & (_TW - 1)
                base_m = (fm << 20) + x0w
                idx_v[0, sl] = base_m + ((y0 & (_TH - 1)) << 10)
                idx_v[1, sl] = base_m + (((y0 + 1) & (_TH - 1)) << 10)

            ctop = pltpu.async_copy(tab_hbm.at[idx_v.at[0]], top_v, sem_g)
            cbot = pltpu.async_copy(tab_hbm.at[idx_v.at[1]], bot_v, sem_g)
            ctop.wait()
            cbot.wait()

            @plsc.parallel_loop(0, _P // 4, unroll=4)
            def _(j):
                rows = grp + (4 * j)
                ch1 = ch + 4
                t00 = plsc.load_gather(top_v, [rows, ch])
                t01 = plsc.load_gather(top_v, [rows, ch1])
                t10 = plsc.load_gather(bot_v, [rows, ch])
                t11 = plsc.load_gather(bot_v, [rows, ch1])
                fx = plsc.load_gather(fx_v, [rows])
                fy = plsc.load_gather(fy_v, [rows])
                omx = 1.0 - fx
                top = t00 * omx + t01 * fx
                bot = t10 * omx + t11 * fx
                plsc.store_scatter(o_v, [rows >> wsh, rows & (w - 1), ch],
                                   top * (1.0 - fy) + bot * fy)

            pltpu.sync_copy(o_v, out_hbm.at[bi, pl.ds(hi, rpc)])
            return ()

        lax.fori_loop(0, n_chunks, chunk_body, ())

    return k(uvflat, fmflat, table)


def kernel(uv, f_mat, tex0, tex1, tex2, tex3):
    b, h, w, _ = uv.shape
    n = b * h * w
    pairs = [
        jnp.concatenate([t, jnp.roll(t, -1, axis=1)], axis=-1)
        .reshape(_TH * _TW, 2 * _C)
        for t in (tex0, tex1, tex2, tex3)
    ]
    table = jnp.concatenate(pairs, axis=0)
    return _sc_sample(uv.reshape(2 * n), f_mat.reshape(n), table, b, h, w)


# speedup vs baseline: 1.9962x; 1.8130x over previous
import functools

import jax
import jax.numpy as jnp
from jax import lax
from jax.experimental import pallas as pl
from jax.experimental.pallas import tpu as pltpu
from jax.experimental.pallas import tpu_sc as plsc

_T = 4
_TH = _TW = 1024
_C = 4
_L = 16
_WL = 128

_P = 4096


def _sc_sample(uv5, fm5, table, b, h, w):
    n = b * h * w
    info = plsc.get_sparse_core_info()
    nw = info.num_cores * info.num_subcores
    per_w = n // nw
    rows_w = per_w // w
    rpc = _P // w
    n_chunks = per_w // _P
    nwt = w // _WL
    mesh = plsc.VectorSubcoreMesh(core_axis_name="c", subcore_axis_name="s")

    @functools.partial(
        pl.kernel,
        mesh=mesh,
        out_type=jax.ShapeDtypeStruct((b, h, nwt, _C, _WL), jnp.float32),
        compiler_params=pltpu.CompilerParams(
            needs_layout_passes=False, use_tc_tiling_on_sc=False),
        scratch_types=[
            pltpu.VMEM((rpc, nwt, 2, _WL), jnp.float32),
            pltpu.VMEM((nwt, 8, _WL), jnp.int32),
            pltpu.VMEM((2, _P), jnp.int32),
            pltpu.VMEM((_P, 2 * _C), jnp.float32),
            pltpu.VMEM((_P, 2 * _C), jnp.float32),
            pltpu.VMEM((rpc, nwt, _C, _WL), jnp.float32),
            pltpu.SemaphoreType.DMA,
            pltpu.SemaphoreType.DMA,
        ],
    )
    def k(uv_hbm, fm_hbm, tab_hbm, out_hbm,
          uv_v, fm_v, idx_v, top_v, bot_v, o_v, sem_in, sem_g):
        wid = lax.axis_index("s") * info.num_cores + lax.axis_index("c")
        lanes = lax.iota(jnp.int32, _L)

        def frac(u_ref_load, scale):
            x = u_ref_load * scale - 0.5
            xt = x.astype(jnp.int32)
            x0 = jnp.where(x < xt.astype(jnp.float32), xt - 1, xt)
            return x0, x - x0.astype(jnp.float32)

        def chunk_body(kc, _):
            r0 = wid * rows_w + kc * rpc
            bi = r0 // h
            h0 = r0 % h
            cin = [
                pltpu.async_copy(uv_hbm.at[bi, pl.ds(h0, rpc)], uv_v, sem_in),
                pltpu.async_copy(fm_hbm.at[bi, h0 // 8], fm_v, sem_in),
            ]
            for c in cin:
                c.wait()

            @plsc.parallel_loop(0, _P // _L, unroll=4)
            def _(i):
                hs = i >> 5
                wt = (i >> 3) & 3
                wl0 = (i & 7) << 4
                csl = pl.ds(wl0, _L)
                uu = uv_v[hs, wt, 0, csl]
                vv = uv_v[hs, wt, 1, csl]
                fm = fm_v[wt, hs, csl]
                x0, _fx = frac(uu, float(_TW))
                y0, _fy = frac(vv, float(_TH))
                base_m = (fm << 20) + (x0 & (_TW - 1))
                sl = pl.ds(i * _L, _L)
                idx_v[0, sl] = base_m + ((y0 & (_TH - 1)) << 10)
                idx_v[1, sl] = base_m + (((y0 + 1) & (_TH - 1)) << 10)

            ctop = pltpu.async_copy(tab_hbm.at[idx_v.at[0]], top_v, sem_g)
            cbot = pltpu.async_copy(tab_hbm.at[idx_v.at[1]], bot_v, sem_g)
            ctop.wait()
            cbot.wait()

            @plsc.parallel_loop(0, _P // _L, unroll=2)
            def _(i):
                hs = i >> 5
                wt = (i >> 3) & 3
                wl0 = (i & 7) << 4
                csl = pl.ds(wl0, _L)
                uu = uv_v[hs, wt, 0, csl]
                vv = uv_v[hs, wt, 1, csl]
                _x0, fx = frac(uu, float(_TW))
                _y0, fy = frac(vv, float(_TH))
                omx = 1.0 - fx
                omy = 1.0 - fy
                p16 = i * _L + lanes
                for c in range(_C):
                    cc = jnp.full((_L,), c, jnp.int32)
                    cc1 = jnp.full((_L,), c + 4, jnp.int32)
                    t00 = plsc.load_gather(top_v, [p16, cc])
                    t01 = plsc.load_gather(top_v, [p16, cc1])
                    t10 = plsc.load_gather(bot_v, [p16, cc])
                    t11 = plsc.load_gather(bot_v, [p16, cc1])
                    top = t00 * omx + t01 * fx
                    bot = t10 * omx + t11 * fx
                    o_v[hs, wt, c, csl] = top * omy + bot * fy

            pltpu.sync_copy(o_v, out_hbm.at[bi, pl.ds(h0, rpc)])
            return ()

        lax.fori_loop(0, n_chunks, chunk_body, ())

    return k(uv5, fm5, table)


def kernel(uv, f_mat, tex0, tex1, tex2, tex3):
    b, h, w, _ = uv.shape
    pairs = [
        jnp.concatenate([t, jnp.roll(t, -1, axis=1)], axis=-1)
        .reshape(_TH * _TW, 2 * _C)
        for t in (tex0, tex1, tex2, tex3)
    ]
    table = jnp.concatenate(pairs, axis=0)
    uv5 = uv.reshape(b, h, w // _WL, _WL, 2).transpose(0, 1, 2, 4, 3)
    fm5 = f_mat.reshape(b, h // 8, 8, w // _WL, _WL).transpose(0, 1, 3, 2, 4)
    out5 = _sc_sample(uv5, fm5, table, b, h, w)
    return out5.transpose(0, 1, 2, 4, 3).reshape(b, h, w, _C)


# speedup vs baseline: 12.9423x; 6.4834x over previous
import functools

import jax
import jax.numpy as jnp
from jax import lax
from jax.experimental import pallas as pl
from jax.experimental.pallas import tpu as pltpu
from jax.experimental.pallas import tpu_sc as plsc

_T = 4
_TH = _TW = 1024
_C = 4
_L = 16
_WL = 128

_P = 4096
_RPB = 4


def _sc_build_table(tex5s):
    info = plsc.get_sparse_core_info()
    mesh = plsc.VectorSubcoreMesh(core_axis_name="c", subcore_axis_name="s")
    blocks = _TH // 8 // _RPB

    @functools.partial(
        pl.kernel,
        mesh=mesh,
        out_type=jax.ShapeDtypeStruct((_T * _TH * _TW, 2 * _C), jnp.float32),
        compiler_params=pltpu.CompilerParams(
            needs_layout_passes=False, use_tc_tiling_on_sc=False),
        scratch_types=[
            pltpu.VMEM((_RPB, _TW // _WL, _C, _WL), jnp.float32),
            pltpu.VMEM((_RPB * _TW, 2 * _C), jnp.float32),
            pltpu.SemaphoreType.DMA,
        ],
    )
    def bk(t0, t1, t2, t3, tab, in_v, out_v, sem):
        wid = lax.axis_index("s") * info.num_cores + lax.axis_index("c")
        m = wid >> 3
        hb = (wid & 7) * (_TH // 8)
        lanes = lax.iota(jnp.int32, _L)

        def build(tex_ref, mm):
            def blk_body(bi, _):
                h0 = hb + bi * _RPB
                pltpu.async_copy(tex_ref.at[pl.ds(h0, _RPB)], in_v,
                                 sem).wait()

                @plsc.parallel_loop(0, _RPB * (_TW // _L), unroll=4)
                def _(i):
                    r = i >> 6
                    g = i & 63
                    wt = g >> 3
                    wl0 = (g & 7) << 4
                    p16 = (r << 10) + (g << 4) + lanes
                    pm1 = (r << 10) + (((g << 4) + lanes - 1) & (_TW - 1))
                    for c in range(_C):
                        t = in_v[r, wt, c, pl.ds(wl0, _L)]
                        plsc.store_scatter(
                            out_v, [p16, jnp.full((_L,), c, jnp.int32)], t)
                        plsc.store_scatter(
                            out_v, [pm1, jnp.full((_L,), c + 4, jnp.int32)], t)

                row0 = pl.multiple_of(mm * (_TH * _TW) + h0 * _TW,
                                      _RPB * _TW)
                pltpu.sync_copy(out_v, tab.at[pl.ds(row0, _RPB * _TW)])
                return ()

            lax.fori_loop(0, blocks, blk_body, ())

        for mm, tex_ref in enumerate((t0, t1, t2, t3)):
            @pl.when(m == mm)
            def _(tex_ref=tex_ref, mm=mm):
                build(tex_ref, mm)

    return bk(*tex5s)


def _sc_sample(uv5, fm5, table, b, h, w):
    n = b * h * w
    info = plsc.get_sparse_core_info()
    nw = info.num_cores * info.num_subcores
    per_w = n // nw
    rows_w = per_w // w
    rpc = _P // w
    n_chunks = per_w // _P
    nwt = w // _WL
    mesh = plsc.VectorSubcoreMesh(core_axis_name="c", subcore_axis_name="s")

    @functools.partial(
        pl.kernel,
        mesh=mesh,
        out_type=jax.ShapeDtypeStruct((b, h, nwt, _C, _WL), jnp.float32),
        compiler_params=pltpu.CompilerParams(
            needs_layout_passes=False, use_tc_tiling_on_sc=False),
        scratch_types=[
            pltpu.VMEM((rpc, nwt, 2, _WL), jnp.float32),
            pltpu.VMEM((nwt, 8, _WL), jnp.int32),
            pltpu.VMEM((2, _P), jnp.int32),
            pltpu.VMEM((_P, 2 * _C), jnp.float32),
            pltpu.VMEM((_P, 2 * _C), jnp.float32),
            pltpu.VMEM((rpc, nwt, _C, _WL), jnp.float32),
            pltpu.SemaphoreType.DMA,
            pltpu.SemaphoreType.DMA,
        ],
    )
    def k(uv_hbm, fm_hbm, tab_hbm, out_hbm,
          uv_v, fm_v, idx_v, top_v, bot_v, o_v, sem_in, sem_g):
        wid = lax.axis_index("s") * info.num_cores + lax.axis_index("c")
        lanes = lax.iota(jnp.int32, _L)

        def frac(u_ref_load, scale):
            x = u_ref_load * scale - 0.5
            xt = x.astype(jnp.int32)
            x0 = jnp.where(x < xt.astype(jnp.float32), xt - 1, xt)
            return x0, x - x0.astype(jnp.float32)

        def chunk_body(kc, _):
            r0 = wid * rows_w + kc * rpc
            bi = r0 // h
            h0 = r0 % h
            cin = [
                pltpu.async_copy(uv_hbm.at[bi, pl.ds(h0, rpc)], uv_v, sem_in),
                pltpu.async_copy(fm_hbm.at[bi, h0 // 8], fm_v, sem_in),
            ]
            for c in cin:
                c.wait()

            @plsc.parallel_loop(0, _P // _L, unroll=4)
            def _(i):
                hs = i >> 5
                wt = (i >> 3) & 3
                wl0 = (i & 7) << 4
                csl = pl.ds(wl0, _L)
                uu = uv_v[hs, wt, 0, csl]
                vv = uv_v[hs, wt, 1, csl]
                fm = fm_v[wt, hs, csl]
                x0, _fx = frac(uu, float(_TW))
                y0, _fy = frac(vv, float(_TH))
                base_m = (fm << 20) + (x0 & (_TW - 1))
                sl = pl.ds(i * _L, _L)
                idx_v[0, sl] = base_m + ((y0 & (_TH - 1)) << 10)
                idx_v[1, sl] = base_m + (((y0 + 1) & (_TH - 1)) << 10)

            ctop = pltpu.async_copy(tab_hbm.at[idx_v.at[0]], top_v, sem_g)
            cbot = pltpu.async_copy(tab_hbm.at[idx_v.at[1]], bot_v, sem_g)
            ctop.wait()
            cbot.wait()

            @plsc.parallel_loop(0, _P // _L, unroll=2)
            def _(i):
                hs = i >> 5
                wt = (i >> 3) & 3
                wl0 = (i & 7) << 4
                csl = pl.ds(wl0, _L)
                uu = uv_v[hs, wt, 0, csl]
                vv = uv_v[hs, wt, 1, csl]
                _x0, fx = frac(uu, float(_TW))
                _y0, fy = frac(vv, float(_TH))
                omx = 1.0 - fx
                omy = 1.0 - fy
                p16 = i * _L + lanes
                for c in range(_C):
                    cc = jnp.full((_L,), c, jnp.int32)
                    cc1 = jnp.full((_L,), c + 4, jnp.int32)
                    t00 = plsc.load_gather(top_v, [p16, cc])
                    t01 = plsc.load_gather(top_v, [p16, cc1])
                    t10 = plsc.load_gather(bot_v, [p16, cc])
                    t11 = plsc.load_gather(bot_v, [p16, cc1])
                    top = t00 * omx + t01 * fx
                    bot = t10 * omx + t11 * fx
                    o_v[hs, wt, c, csl] = top * omy + bot * fy

            pltpu.sync_copy(o_v, out_hbm.at[bi, pl.ds(h0, rpc)])
            return ()

        lax.fori_loop(0, n_chunks, chunk_body, ())

    return k(uv5, fm5, table)


def kernel(uv, f_mat, tex0, tex1, tex2, tex3):
    b, h, w, _ = uv.shape
    tex5s = [
        t.reshape(_TH, _TW // _WL, _WL, _C).transpose(0, 1, 3, 2)
        for t in (tex0, tex1, tex2, tex3)
    ]
    table = _sc_build_table(tex5s)
    uv5 = uv.reshape(b, h, w // _WL, _WL, 2).transpose(0, 1, 2, 4, 3)
    fm5 = f_mat.reshape(b, h // 8, 8, w // _WL, _WL).transpose(0, 1, 3, 2, 4)
    out5 = _sc_sample(uv5, fm5, table, b, h, w)
    return out5.transpose(0, 1, 2, 4, 3).reshape(b, h, w, _C)


# speedup vs baseline: 14.9410x; 1.1544x over previous
import functools

import jax
import jax.numpy as jnp
from jax import lax
from jax.experimental import pallas as pl
from jax.experimental.pallas import tpu as pltpu
from jax.experimental.pallas import tpu_sc as plsc

_T = 4
_TH = _TW = 1024
_C = 4
_L = 16
_WL = 128

_P = 2048
_RPB = 4


def _sc_build_table(tex5s):
    info = plsc.get_sparse_core_info()
    mesh = plsc.VectorSubcoreMesh(core_axis_name="c", subcore_axis_name="s")
    blocks = _TH // 8 // _RPB

    @functools.partial(
        pl.kernel,
        mesh=mesh,
        out_type=jax.ShapeDtypeStruct((_T * _TH * _TW, 2 * _C), jnp.float32),
        compiler_params=pltpu.CompilerParams(
            needs_layout_passes=False, use_tc_tiling_on_sc=False),
        scratch_types=[
            pltpu.VMEM((_RPB, _TW // _WL, _C, _WL), jnp.float32),
            pltpu.VMEM((_RPB * _TW, 2 * _C), jnp.float32),
            pltpu.SemaphoreType.DMA,
        ],
    )
    def bk(t0, t1, t2, t3, tab, in_v, out_v, sem):
        wid = lax.axis_index("s") * info.num_cores + lax.axis_index("c")
        m = wid >> 3
        hb = (wid & 7) * (_TH // 8)
        lanes = lax.iota(jnp.int32, _L)

        def build(tex_ref, mm):
            def blk_body(bi, _):
                h0 = hb + bi * _RPB
                pltpu.async_copy(tex_ref.at[pl.ds(h0, _RPB)], in_v,
                                 sem).wait()

                @plsc.parallel_loop(0, _RPB * (_TW // _L), unroll=4)
                def _(i):
                    r = i >> 6
                    g = i & 63
                    wt = g >> 3
                    wl0 = (g & 7) << 4
                    p16 = (r << 10) + (g << 4) + lanes
                    pm1 = (r << 10) + (((g << 4) + lanes - 1) & (_TW - 1))
                    for c in range(_C):
                        t = in_v[r, wt, c, pl.ds(wl0, _L)]
                        plsc.store_scatter(
                            out_v, [p16, jnp.full((_L,), c, jnp.int32)], t)
                        plsc.store_scatter(
                            out_v, [pm1, jnp.full((_L,), c + 4, jnp.int32)], t)

                row0 = pl.multiple_of(mm * (_TH * _TW) + h0 * _TW,
                                      _RPB * _TW)
                pltpu.sync_copy(out_v, tab.at[pl.ds(row0, _RPB * _TW)])
                return ()

            lax.fori_loop(0, blocks, blk_body, ())

        for mm, tex_ref in enumerate((t0, t1, t2, t3)):
            @pl.when(m == mm)
            def _(tex_ref=tex_ref, mm=mm):
                build(tex_ref, mm)

    return bk(*tex5s)


def _sc_sample(uv5, fm5, table, b, h, w):
    n = b * h * w
    info = plsc.get_sparse_core_info()
    nw = info.num_cores * info.num_subcores
    per_w = n // nw
    rows_w = per_w // w
    rpc = _P // w
    n_chunks = per_w // _P
    nwt = w // _WL
    mesh = plsc.VectorSubcoreMesh(core_axis_name="c", subcore_axis_name="s")

    @functools.partial(
        pl.kernel,
        mesh=mesh,
        out_type=jax.ShapeDtypeStruct((b, h, nwt, _C, _WL), jnp.float32),
        compiler_params=pltpu.CompilerParams(
            needs_layout_passes=False, use_tc_tiling_on_sc=False),
        scratch_types=[
            pltpu.VMEM((2, rpc, nwt, 2, _WL), jnp.float32),
            pltpu.VMEM((2, nwt, rpc, _WL), jnp.int32),
            pltpu.VMEM((2, 2, _P), jnp.int32),
            pltpu.VMEM((2, _P, 2 * _C), jnp.float32),
            pltpu.VMEM((2, _P, 2 * _C), jnp.float32),
            pltpu.VMEM((2, rpc, nwt, _C, _WL), jnp.float32),
            pltpu.SemaphoreType.DMA,
            pltpu.SemaphoreType.DMA,
        ],
    )
    def k(uv_hbm, fm_hbm, tab_hbm, out_hbm,
          uv_v, fm_v, idx_v, top_v, bot_v, o_v, sem_in, sem_g):
        wid = lax.axis_index("s") * info.num_cores + lax.axis_index("c")
        lanes = lax.iota(jnp.int32, _L)

        def frac(val, scale):
            x = val * scale - 0.5
            xt = x.astype(jnp.int32)
            x0 = jnp.where(x < xt.astype(jnp.float32), xt - 1, xt)
            return x0, x - x0.astype(jnp.float32)

        def coords(kc):
            r0 = wid * rows_w + kc * rpc
            return r0 // h, r0 % h

        def issue_in(kc, s):
            bi, h0 = coords(kc)
            pltpu.async_copy(uv_hbm.at[bi, pl.ds(h0, rpc)], uv_v.at[s],
                             sem_in)
            pltpu.async_copy(fm_hbm.at[bi, h0 // 8, :, pl.ds(h0 % 8, rpc)],
                             fm_v.at[s], sem_in)

        def wait_in(kc, s):
            bi, h0 = coords(kc)
            pltpu.make_async_copy(uv_hbm.at[bi, pl.ds(h0, rpc)], uv_v.at[s],
                                  sem_in).wait()
            pltpu.make_async_copy(fm_hbm.at[bi, h0 // 8, :,
                                            pl.ds(h0 % 8, rpc)],
                                  fm_v.at[s], sem_in).wait()

        def phase2_and_fire(s):
            @plsc.parallel_loop(0, _P // _L, unroll=4)
            def _(i):
                hs = i >> 5
                wt = (i >> 3) & 3
                wl0 = (i & 7) << 4
                csl = pl.ds(wl0, _L)
                uu = uv_v[s, hs, wt, 0, csl]
                vv = uv_v[s, hs, wt, 1, csl]
                fm = fm_v[s, wt, hs, csl]
                x0, _fx = frac(uu, float(_TW))
                y0, _fy = frac(vv, float(_TH))
                base_m = (fm << 20) + (x0 & (_TW - 1))
                sl = pl.ds(i * _L, _L)
                idx_v[s, 0, sl] = base_m + ((y0 & (_TH - 1)) << 10)
                idx_v[s, 1, sl] = base_m + (((y0 + 1) & (_TH - 1)) << 10)

            pltpu.async_copy(tab_hbm.at[idx_v.at[s, 0]], top_v.at[s], sem_g)
            pltpu.async_copy(tab_hbm.at[idx_v.at[s, 1]], bot_v.at[s], sem_g)

        def combine_and_out(kc, s):
            pltpu.make_async_copy(tab_hbm.at[idx_v.at[s, 0]], top_v.at[s],
                                  sem_g).wait()
            pltpu.make_async_copy(tab_hbm.at[idx_v.at[s, 1]], bot_v.at[s],
                                  sem_g).wait()

            @plsc.parallel_loop(0, _P // _L, unroll=2)
            def _(i):
                hs = i >> 5
                wt = (i >> 3) & 3
                wl0 = (i & 7) << 4
                csl = pl.ds(wl0, _L)
                uu = uv_v[s, hs, wt, 0, csl]
                vv = uv_v[s, hs, wt, 1, csl]
                _x0, fx = frac(uu, float(_TW))
                _y0, fy = frac(vv, float(_TH))
                omx = 1.0 - fx
                omy = 1.0 - fy
                p16 = i * _L + lanes
                for c in range(_C):
                    cc = jnp.full((_L,), c, jnp.int32)
                    cc1 = jnp.full((_L,), c + 4, jnp.int32)
                    t00 = plsc.load_gather(top_v.at[s], [p16, cc])
                    t01 = plsc.load_gather(top_v.at[s], [p16, cc1])
                    t10 = plsc.load_gather(bot_v.at[s], [p16, cc])
                    t11 = plsc.load_gather(bot_v.at[s], [p16, cc1])
                    top = t00 * omx + t01 * fx
                    bot = t10 * omx + t11 * fx
                    o_v[s, hs, wt, c, csl] = top * omy + bot * fy

            bi, h0 = coords(kc)
            pltpu.sync_copy(o_v.at[s], out_hbm.at[bi, pl.ds(h0, rpc)])

        issue_in(0, 0)

        def chunk_body(kc, _):
            s = kc & 1
            wait_in(kc, s)
            phase2_and_fire(s)

            @pl.when(kc > 0)
            def _():
                combine_and_out(kc - 1, 1 - s)

            @pl.when(kc + 1 < n_chunks)
            def _():
                issue_in(kc + 1, 1 - s)
            return ()

        lax.fori_loop(0, n_chunks, chunk_body, ())
        combine_and_out(n_chunks - 1, (n_chunks - 1) & 1)

    return k(uv5, fm5, table)


def kernel(uv, f_mat, tex0, tex1, tex2, tex3):
    b, h, w, _ = uv.shape
    tex5s = [
        t.reshape(_TH, _TW // _WL, _WL, _C).transpose(0, 1, 3, 2)
        for t in (tex0, tex1, tex2, tex3)
    ]
    table = _sc_build_table(tex5s)
    uv5 = uv.reshape(b, h, w // _WL, _WL, 2).transpose(0, 1, 2, 4, 3)
    fm5 = f_mat.reshape(b, h // 8, 8, w // _WL, _WL).transpose(0, 1, 3, 2, 4)
    out5 = _sc_sample(uv5, fm5, table, b, h, w)
    return out5.transpose(0, 1, 2, 4, 3).reshape(b, h, w, _C)


# speedup vs baseline: 19.6648x; 1.3162x over previous
import functools

import jax
import jax.numpy as jnp
from jax import lax
from jax.experimental import pallas as pl
from jax.experimental.pallas import tpu as pltpu
from jax.experimental.pallas import tpu_sc as plsc

_T = 4
_TH = _TW = 1024
_C = 4
_L = 16
_WL = 128

_P = 2048
_RPB = 4


def _sc_build_table(tex5s):
    info = plsc.get_sparse_core_info()
    mesh = plsc.VectorSubcoreMesh(core_axis_name="c", subcore_axis_name="s")
    blocks = _TH // 8 // _RPB

    @functools.partial(
        pl.kernel,
        mesh=mesh,
        out_type=jax.ShapeDtypeStruct((_T * _TH * _TW, 2 * _C), jnp.float32),
        compiler_params=pltpu.CompilerParams(
            needs_layout_passes=False, use_tc_tiling_on_sc=False),
        scratch_types=[
            pltpu.VMEM((2, _RPB, _TW // _WL, _C, _WL), jnp.float32),
            pltpu.VMEM((2, _RPB * _TW, 2 * _C), jnp.float32),
            pltpu.SemaphoreType.DMA,
            pltpu.SemaphoreType.DMA,
        ],
    )
    def bk(t0, t1, t2, t3, tab, in_v, out_v, sem_in, sem_out):
        wid = lax.axis_index("s") * info.num_cores + lax.axis_index("c")
        m = wid >> 3
        hb = (wid & 7) * (_TH // 8)
        lanes = lax.iota(jnp.int32, _L)

        def build(tex_ref, mm):
            def tab_slice(bi):
                h0 = hb + bi * _RPB
                row0 = pl.multiple_of(mm * (_TH * _TW) + h0 * _TW,
                                      _RPB * _TW)
                return tab.at[pl.ds(row0, _RPB * _TW)]

            def issue_in(bi, s):
                pltpu.async_copy(tex_ref.at[pl.ds(hb + bi * _RPB, _RPB)],
                                 in_v.at[s], sem_in)

            def wait_in(bi, s):
                pltpu.make_async_copy(
                    tex_ref.at[pl.ds(hb + bi * _RPB, _RPB)], in_v.at[s],
                    sem_in).wait()

            def transpose(s):
                @plsc.parallel_loop(0, _RPB * (_TW // _L), unroll=4)
                def _(i):
                    r = i >> 6
                    g = i & 63
                    wt = g >> 3
                    wl0 = (g & 7) << 4
                    p16 = (r << 10) + (g << 4) + lanes
                    pm1 = (r << 10) + (((g << 4) + lanes - 1) & (_TW - 1))
                    for c in range(_C):
                        t = in_v[s, r, wt, c, pl.ds(wl0, _L)]
                        plsc.store_scatter(
                            out_v.at[s],
                            [p16, jnp.full((_L,), c, jnp.int32)], t)
                        plsc.store_scatter(
                            out_v.at[s],
                            [pm1, jnp.full((_L,), c + 4, jnp.int32)], t)

            issue_in(0, 0)

            def blk_body(bi, _):
                s = bi & 1

                @pl.when(bi + 1 < blocks)
                def _():
                    issue_in(bi + 1, 1 - s)

                wait_in(bi, s)

                @pl.when(bi > 1)
                def _():
                    pltpu.make_async_copy(out_v.at[s], tab_slice(bi - 2),
                                          sem_out).wait()

                transpose(s)
                pltpu.async_copy(out_v.at[s], tab_slice(bi), sem_out)
                return ()

            lax.fori_loop(0, blocks, blk_body, ())
            for tail in (blocks - 2, blocks - 1):
                pltpu.make_async_copy(out_v.at[tail & 1], tab_slice(tail),
                                      sem_out).wait()

        for mm, tex_ref in enumerate((t0, t1, t2, t3)):
            @pl.when(m == mm)
            def _(tex_ref=tex_ref, mm=mm):
                build(tex_ref, mm)

    return bk(*tex5s)


def _sc_sample(uv5, fm5, table, b, h, w):
    n = b * h * w
    info = plsc.get_sparse_core_info()
    nw = info.num_cores * info.num_subcores
    per_w = n // nw
    rows_w = per_w // w
    rpc = _P // w
    n_chunks = per_w // _P
    nwt = w // _WL
    mesh = plsc.VectorSubcoreMesh(core_axis_name="c", subcore_axis_name="s")

    @functools.partial(
        pl.kernel,
        mesh=mesh,
        out_type=jax.ShapeDtypeStruct((b, h, nwt, _C, _WL), jnp.float32),
        compiler_params=pltpu.CompilerParams(
            needs_layout_passes=False, use_tc_tiling_on_sc=False),
        scratch_types=[
            pltpu.VMEM((2, rpc, nwt, 2, _WL), jnp.float32),
            pltpu.VMEM((2, nwt, rpc, _WL), jnp.int32),
            pltpu.VMEM((2, 2, _P), jnp.int32),
            pltpu.VMEM((2, _P, 2 * _C), jnp.float32),
            pltpu.VMEM((2, _P, 2 * _C), jnp.float32),
            pltpu.VMEM((2, rpc, nwt, _C, _WL), jnp.float32),
            pltpu.SemaphoreType.DMA,
            pltpu.SemaphoreType.DMA,
        ],
    )
    def k(uv_hbm, fm_hbm, tab_hbm, out_hbm,
          uv_v, fm_v, idx_v, top_v, bot_v, o_v, sem_in, sem_g):
        wid = lax.axis_index("s") * info.num_cores + lax.axis_index("c")
        lanes = lax.iota(jnp.int32, _L)

        def frac(val, scale):
            x = val * scale - 0.5
            xt = x.astype(jnp.int32)
            x0 = jnp.where(x < xt.astype(jnp.float32), xt - 1, xt)
            return x0, x - x0.astype(jnp.float32)

        def coords(kc):
            r0 = wid * rows_w + kc * rpc
            return r0 // h, r0 % h

        def issue_in(kc, s):
            bi, h0 = coords(kc)
            pltpu.async_copy(uv_hbm.at[bi, pl.ds(h0, rpc)], uv_v.at[s],
                             sem_in)
            pltpu.async_copy(fm_hbm.at[bi, h0 // 8, :, pl.ds(h0 % 8, rpc)],
                             fm_v.at[s], sem_in)

        def wait_in(kc, s):
            bi, h0 = coords(kc)
            pltpu.make_async_copy(uv_hbm.at[bi, pl.ds(h0, rpc)], uv_v.at[s],
                                  sem_in).wait()
            pltpu.make_async_copy(fm_hbm.at[bi, h0 // 8, :,
                                            pl.ds(h0 % 8, rpc)],
                                  fm_v.at[s], sem_in).wait()

        def phase2_and_fire(s):
            @plsc.parallel_loop(0, _P // _L, unroll=4)
            def _(i):
                hs = i >> 5
                wt = (i >> 3) & 3
                wl0 = (i & 7) << 4
                csl = pl.ds(wl0, _L)
                uu = uv_v[s, hs, wt, 0, csl]
                vv = uv_v[s, hs, wt, 1, csl]
                fm = fm_v[s, wt, hs, csl]
                x0, _fx = frac(uu, float(_TW))
                y0, _fy = frac(vv, float(_TH))
                base_m = (fm << 20) + (x0 & (_TW - 1))
                sl = pl.ds(i * _L, _L)
                idx_v[s, 0, sl] = base_m + ((y0 & (_TH - 1)) << 10)
                idx_v[s, 1, sl] = base_m + (((y0 + 1) & (_TH - 1)) << 10)

            pltpu.async_copy(tab_hbm.at[idx_v.at[s, 0]], top_v.at[s], sem_g)
            pltpu.async_copy(tab_hbm.at[idx_v.at[s, 1]], bot_v.at[s], sem_g)

        def combine_and_out(kc, s):
            pltpu.make_async_copy(tab_hbm.at[idx_v.at[s, 0]], top_v.at[s],
                                  sem_g).wait()
            pltpu.make_async_copy(tab_hbm.at[idx_v.at[s, 1]], bot_v.at[s],
                                  sem_g).wait()

            @plsc.parallel_loop(0, _P // _L, unroll=2)
            def _(i):
                hs = i >> 5
                wt = (i >> 3) & 3
                wl0 = (i & 7) << 4
                csl = pl.ds(wl0, _L)
                uu = uv_v[s, hs, wt, 0, csl]
                vv = uv_v[s, hs, wt, 1, csl]
                _x0, fx = frac(uu, float(_TW))
                _y0, fy = frac(vv, float(_TH))
                omx = 1.0 - fx
                omy = 1.0 - fy
                p16 = i * _L + lanes
                for c in range(_C):
                    cc = jnp.full((_L,), c, jnp.int32)
                    cc1 = jnp.full((_L,), c + 4, jnp.int32)
                    t00 = plsc.load_gather(top_v.at[s], [p16, cc])
                    t01 = plsc.load_gather(top_v.at[s], [p16, cc1])
                    t10 = plsc.load_gather(bot_v.at[s], [p16, cc])
                    t11 = plsc.load_gather(bot_v.at[s], [p16, cc1])
                    top = t00 * omx + t01 * fx
                    bot = t10 * omx + t11 * fx
                    o_v[s, hs, wt, c, csl] = top * omy + bot * fy

            bi, h0 = coords(kc)
            pltpu.sync_copy(o_v.at[s], out_hbm.at[bi, pl.ds(h0, rpc)])

        issue_in(0, 0)

        def chunk_body(kc, _):
            s = kc & 1
            wait_in(kc, s)
            phase2_and_fire(s)

            @pl.when(kc > 0)
            def _():
                combine_and_out(kc - 1, 1 - s)

            @pl.when(kc + 1 < n_chunks)
            def _():
                issue_in(kc + 1, 1 - s)
            return ()

        lax.fori_loop(0, n_chunks, chunk_body, ())
        combine_and_out(n_chunks - 1, (n_chunks - 1) & 1)

    return k(uv5, fm5, table)


def kernel(uv, f_mat, tex0, tex1, tex2, tex3):
    b, h, w, _ = uv.shape
    tex5s = [
        t.reshape(_TH, _TW // _WL, _WL, _C).transpose(0, 1, 3, 2)
        for t in (tex0, tex1, tex2, tex3)
    ]
    table = _sc_build_table(tex5s)
    uv5 = uv.reshape(b, h, w // _WL, _WL, 2).transpose(0, 1, 2, 4, 3)
    fm5 = f_mat.reshape(b, h // 8, 8, w // _WL, _WL).transpose(0, 1, 3, 2, 4)
    out5 = _sc_sample(uv5, fm5, table, b, h, w)
    return out5.transpose(0, 1, 2, 4, 3).reshape(b, h, w, _C)


# speedup vs baseline: 20.1730x; 1.0258x over previous
import functools

import jax
import jax.numpy as jnp
from jax import lax
from jax.experimental import pallas as pl
from jax.experimental.pallas import tpu as pltpu
from jax.experimental.pallas import tpu_sc as plsc

_T = 4
_TH = _TW = 1024
_C = 4
_L = 16
_WL = 128

_P = 2048
_RPB = 4


def _sc_build_table(tex5s):
    info = plsc.get_sparse_core_info()
    mesh = plsc.VectorSubcoreMesh(core_axis_name="c", subcore_axis_name="s")
    blocks = _TH // 8 // _RPB

    @functools.partial(
        pl.kernel,
        mesh=mesh,
        out_type=jax.ShapeDtypeStruct((_T * _TH * _TW, 2 * _C), jnp.float32),
        compiler_params=pltpu.CompilerParams(
            needs_layout_passes=False, use_tc_tiling_on_sc=False),
        scratch_types=[
            pltpu.VMEM((2, _RPB, _TW // _WL, _C, _WL), jnp.float32),
            pltpu.VMEM((2, _RPB * _TW, 2 * _C), jnp.float32),
            pltpu.SemaphoreType.DMA,
            pltpu.SemaphoreType.DMA,
        ],
    )
    def bk(t0, t1, t2, t3, tab, in_v, out_v, sem_in, sem_out):
        wid = lax.axis_index("s") * info.num_cores + lax.axis_index("c")
        m = wid >> 3
        hb = (wid & 7) * (_TH // 8)
        lanes = lax.iota(jnp.int32, _L)

        def build(tex_ref, mm):
            def tab_slice(bi):
                h0 = hb + bi * _RPB
                row0 = pl.multiple_of(mm * (_TH * _TW) + h0 * _TW,
                                      _RPB * _TW)
                return tab.at[pl.ds(row0, _RPB * _TW)]

            def issue_in(bi, s):
                pltpu.async_copy(tex_ref.at[pl.ds(hb + bi * _RPB, _RPB)],
                                 in_v.at[s], sem_in)

            def wait_in(bi, s):
                pltpu.make_async_copy(
                    tex_ref.at[pl.ds(hb + bi * _RPB, _RPB)], in_v.at[s],
                    sem_in).wait()

            def transpose(s):
                @plsc.parallel_loop(0, _RPB * (_TW // _L), unroll=4)
                def _(i):
                    r = i >> 6
                    g = i & 63
                    wt = g >> 3
                    wl0 = (g & 7) << 4
                    p16 = (r << 10) + (g << 4) + lanes
                    pm1 = (r << 10) + (((g << 4) + lanes - 1) & (_TW - 1))
                    for c in range(_C):
                        t = in_v[s, r, wt, c, pl.ds(wl0, _L)]
                        plsc.store_scatter(
                            out_v.at[s],
                            [p16, jnp.full((_L,), c, jnp.int32)], t)
                        plsc.store_scatter(
                            out_v.at[s],
                            [pm1, jnp.full((_L,), c + 4, jnp.int32)], t)

            issue_in(0, 0)

            def blk_body(bi, _):
                s = bi & 1

                @pl.when(bi + 1 < blocks)
                def _():
                    issue_in(bi + 1, 1 - s)

                wait_in(bi, s)

                @pl.when(bi > 1)
                def _():
                    pltpu.make_async_copy(out_v.at[s], tab_slice(bi - 2),
                                          sem_out).wait()

                transpose(s)
                pltpu.async_copy(out_v.at[s], tab_slice(bi), sem_out)
                return ()

            lax.fori_loop(0, blocks, blk_body, ())
            for tail in (blocks - 2, blocks - 1):
                pltpu.make_async_copy(out_v.at[tail & 1], tab_slice(tail),
                                      sem_out).wait()

        for mm, tex_ref in enumerate((t0, t1, t2, t3)):
            @pl.when(m == mm)
            def _(tex_ref=tex_ref, mm=mm):
                build(tex_ref, mm)

    return bk(*tex5s)


def _sc_sample(uv5, fm5, table, b, h, w):
    n = b * h * w
    info = plsc.get_sparse_core_info()
    nw = info.num_cores * info.num_subcores
    per_w = n // nw
    rows_w = per_w // w
    rpc = _P // w
    n_chunks = per_w // _P
    nwt = w // _WL
    mesh = plsc.VectorSubcoreMesh(core_axis_name="c", subcore_axis_name="s")

    @functools.partial(
        pl.kernel,
        mesh=mesh,
        out_type=jax.ShapeDtypeStruct((b, h, nwt, _C, _WL), jnp.float32),
        compiler_params=pltpu.CompilerParams(
            needs_layout_passes=False, use_tc_tiling_on_sc=False),
        scratch_types=[
            pltpu.VMEM((3, rpc, nwt, 2, _WL), jnp.float32),
            pltpu.VMEM((3, nwt, rpc, _WL), jnp.int32),
            pltpu.VMEM((2, 2, _P), jnp.int32),
            pltpu.VMEM((2, _P, 2 * _C), jnp.float32),
            pltpu.VMEM((2, _P, 2 * _C), jnp.float32),
            pltpu.VMEM((2, rpc, nwt, _C, _WL), jnp.float32),
            pltpu.SemaphoreType.DMA,
            pltpu.SemaphoreType.DMA,
            pltpu.SemaphoreType.DMA,
        ],
    )
    def k(uv_hbm, fm_hbm, tab_hbm, out_hbm,
          uv_v, fm_v, idx_v, top_v, bot_v, o_v, sem_in, sem_g, sem_out):
        wid = lax.axis_index("s") * info.num_cores + lax.axis_index("c")
        lanes = lax.iota(jnp.int32, _L)

        def frac(val, scale):
            x = val * scale - 0.5
            xt = x.astype(jnp.int32)
            x0 = jnp.where(x < xt.astype(jnp.float32), xt - 1, xt)
            return x0, x - x0.astype(jnp.float32)

        def coords(kc):
            r0 = wid * rows_w + kc * rpc
            return r0 // h, r0 % h

        def issue_in(kc, s):
            bi, h0 = coords(kc)
            pltpu.async_copy(uv_hbm.at[bi, pl.ds(h0, rpc)], uv_v.at[s],
                             sem_in)
            pltpu.async_copy(fm_hbm.at[bi, h0 // 8, :, pl.ds(h0 % 8, rpc)],
                             fm_v.at[s], sem_in)

        def wait_in(kc, s):
            bi, h0 = coords(kc)
            pltpu.make_async_copy(uv_hbm.at[bi, pl.ds(h0, rpc)], uv_v.at[s],
                                  sem_in).wait()
            pltpu.make_async_copy(fm_hbm.at[bi, h0 // 8, :,
                                            pl.ds(h0 % 8, rpc)],
                                  fm_v.at[s], sem_in).wait()

        def phase2_and_fire(s3, s):
            @plsc.parallel_loop(0, _P // _L, unroll=4)
            def _(i):
                hs = i >> 5
                wt = (i >> 3) & 3
                wl0 = (i & 7) << 4
                csl = pl.ds(wl0, _L)
                uu = uv_v[s3, hs, wt, 0, csl]
                vv = uv_v[s3, hs, wt, 1, csl]
                fm = fm_v[s3, wt, hs, csl]
                x0, _fx = frac(uu, float(_TW))
                y0, _fy = frac(vv, float(_TH))
                base_m = (fm << 20) + (x0 & (_TW - 1))
                sl = pl.ds(i * _L, _L)
                idx_v[s, 0, sl] = base_m + ((y0 & (_TH - 1)) << 10)
                idx_v[s, 1, sl] = base_m + (((y0 + 1) & (_TH - 1)) << 10)

            pltpu.async_copy(tab_hbm.at[idx_v.at[s, 0]], top_v.at[s], sem_g)
            pltpu.async_copy(tab_hbm.at[idx_v.at[s, 1]], bot_v.at[s], sem_g)

        def combine_and_out(kc, s):
            s3 = kc % 3
            pltpu.make_async_copy(tab_hbm.at[idx_v.at[s, 0]], top_v.at[s],
                                  sem_g).wait()
            pltpu.make_async_copy(tab_hbm.at[idx_v.at[s, 1]], bot_v.at[s],
                                  sem_g).wait()

            @pl.when(kc > 1)
            def _():
                bi2, h02 = coords(kc - 2)
                pltpu.make_async_copy(o_v.at[s], out_hbm.at[bi2,
                                                            pl.ds(h02, rpc)],
                                      sem_out).wait()

            @plsc.parallel_loop(0, _P // _L, unroll=2)
            def _(i):
                hs = i >> 5
                wt = (i >> 3) & 3
                wl0 = (i & 7) << 4
                csl = pl.ds(wl0, _L)
                uu = uv_v[s3, hs, wt, 0, csl]
                vv = uv_v[s3, hs, wt, 1, csl]
                _x0, fx = frac(uu, float(_TW))
                _y0, fy = frac(vv, float(_TH))
                omx = 1.0 - fx
                omy = 1.0 - fy
                p16 = i * _L + lanes
                for c in range(_C):
                    cc = jnp.full((_L,), c, jnp.int32)
                    cc1 = jnp.full((_L,), c + 4, jnp.int32)
                    t00 = plsc.load_gather(top_v.at[s], [p16, cc])
                    t01 = plsc.load_gather(top_v.at[s], [p16, cc1])
                    t10 = plsc.load_gather(bot_v.at[s], [p16, cc])
                    t11 = plsc.load_gather(bot_v.at[s], [p16, cc1])
                    top = t00 * omx + t01 * fx
                    bot = t10 * omx + t11 * fx
                    o_v[s, hs, wt, c, csl] = top * omy + bot * fy

            bi, h0 = coords(kc)
            pltpu.async_copy(o_v.at[s], out_hbm.at[bi, pl.ds(h0, rpc)],
                             sem_out)

        issue_in(0, 0)

        def chunk_body(kc, _):
            s = kc & 1
            s3 = kc % 3
            wait_in(kc, s3)

            @pl.when(kc + 1 < n_chunks)
            def _():
                issue_in(kc + 1, (kc + 1) % 3)

            phase2_and_fire(s3, s)

            @pl.when(kc > 0)
            def _():
                combine_and_out(kc - 1, 1 - s)
            return ()

        lax.fori_loop(0, n_chunks, chunk_body, ())
        combine_and_out(n_chunks - 1, (n_chunks - 1) & 1)
        for tail in (n_chunks - 2, n_chunks - 1):
            bi, h0 = coords(tail)
            pltpu.make_async_copy(o_v.at[tail & 1],
                                  out_hbm.at[bi, pl.ds(h0, rpc)],
                                  sem_out).wait()

    return k(uv5, fm5, table)


def kernel(uv, f_mat, tex0, tex1, tex2, tex3):
    b, h, w, _ = uv.shape
    tex5s = [
        t.reshape(_TH, _TW // _WL, _WL, _C).transpose(0, 1, 3, 2)
        for t in (tex0, tex1, tex2, tex3)
    ]
    table = _sc_build_table(tex5s)
    uv5 = uv.reshape(b, h, w // _WL, _WL, 2).transpose(0, 1, 2, 4, 3)
    fm5 = f_mat.reshape(b, h // 8, 8, w // _WL, _WL).transpose(0, 1, 3, 2, 4)
    out5 = _sc_sample(uv5, fm5, table, b, h, w)
    return out5.transpose(0, 1, 2, 4, 3).reshape(b, h, w, _C)
